# Initial kernel scaffold; baseline (speedup 1.0000x reference)
import jax
import jax.numpy as jnp
from jax.experimental import pallas as pl


def kernel(frame_t, frame_t1, params):
    raise NotImplementedError("write your pallas kernel here")


# speedup vs baseline: 4.7756x; 4.7756x over previous
import functools

import jax
import jax.numpy as jnp
from jax import lax
from jax.experimental import pallas as pl
from jax.experimental.pallas import tpu as pltpu

FRAME_DIM = 256
HIDDEN = 512
ACTION_DIM = 64
NUM_ACTIONS = 256
COMMIT = 0.25


def _layernorm(x, g, b):
    mu = jnp.mean(x, axis=-1, keepdims=True)
    var = jnp.mean((x - mu) ** 2, axis=-1, keepdims=True)
    return (x - mu) / jnp.sqrt(var + 1e-5) * g + b


def _gelu(x):
    return x * 0.5 * (1.0 + lax.erf(x * 0.7071067811865476))


def _dot(a, b):
    return jnp.dot(a, b, preferred_element_type=jnp.float32)


def _fused_kernel(n_rows, tile,
                  ft_ref, ft1_ref,
                  ei_wa, ei_wb, ei_b,
                  e0_w, e0_b, e0_g, e0_beta,
                  e1_w, e1_b, e1_g, e1_beta,
                  e2_w, e2_b, e2_g, e2_beta,
                  eo_w, eo_b,
                  c_ref, ct_ref,
                  pi_wa, pi_wb, pi_b,
                  p0_w, p0_b, p0_g, p0_beta,
                  p1_w, p1_b, p1_g, p1_beta,
                  p2_w, p2_b, p2_g, p2_beta,
                  po_w, po_b,
                  pred_ref, idx_ref, commit_ref, cb_ref, vq_ref, perp_ref,
                  hist_acc, sq_acc):
    step = pl.program_id(0)
    nsteps = pl.num_programs(0)

    @pl.when(step == 0)
    def _init():
        hist_acc[...] = jnp.zeros_like(hist_acc)
        sq_acc[...] = jnp.zeros_like(sq_acc)

    ft = ft_ref[...]
    ft1 = ft1_ref[...]

    x = _dot(ft, ei_wa[...]) + _dot(ft1, ei_wb[...]) + ei_b[...]
    for w, b, g, beta in ((e0_w, e0_b, e0_g, e0_beta),
                          (e1_w, e1_b, e1_g, e1_beta),
                          (e2_w, e2_b, e2_g, e2_beta)):
        x = _dot(x, w[...]) + b[...]
        x = _layernorm(x, g[...], beta[...])
        x = _gelu(x)
    z = _dot(x, eo_w[...]) + eo_b[...]

    ct = ct_ref[...]
    c2 = jnp.sum(ct * ct, axis=0, keepdims=True)
    z2 = jnp.sum(z * z, axis=1, keepdims=True)
    dist = z2 - 2.0 * _dot(z, ct) + c2
    dmin = jnp.min(dist, axis=1, keepdims=True)
    cols = lax.broadcasted_iota(jnp.int32, dist.shape, 1)
    idx = jnp.min(jnp.where(dist == dmin, cols, NUM_ACTIONS), axis=1)
    idx_ref[...] = idx[:, None]
    onehot = (cols == idx[:, None]).astype(jnp.float32)
    quant = _dot(onehot, c_ref[...])

    hist_acc[...] += jnp.sum(onehot, axis=0, keepdims=True)
    diff = z - quant
    sq_acc[...] += jnp.sum(diff * diff)[None, None]

    y = _dot(ft, pi_wa[...]) + _dot(quant, pi_wb[...]) + pi_b[...]
    for w, b, g, beta in ((p0_w, p0_b, p0_g, p0_beta),
                          (p1_w, p1_b, p1_g, p1_beta),
                          (p2_w, p2_b, p2_g, p2_beta)):
        y = _dot(y, w[...]) + b[...]
        y = _layernorm(y, g[...], beta[...])
        y = _gelu(y)
    out = _dot(y, po_w[...]) + po_b[...]
    pred_ref[...] = ft + out

    @pl.when(step == nsteps - 1)
    def _finalize():
        s = sq_acc[...] / (n_rows * ACTION_DIM)
        commit_ref[...] = s
        cb_ref[...] = s
        vq_ref[...] = s + COMMIT * s
        probs = hist_acc[...] / n_rows
        ent = jnp.sum(probs * jnp.log(probs + 1e-10), axis=1, keepdims=True)
        perp_ref[...] = jnp.exp(-ent)


def _run(ft, ft1, flat_weights, tile):
    n_rows = ft.shape[0]
    grid = (n_rows // tile,)

    def row_spec(d):
        return pl.BlockSpec((tile, d), lambda i: (i, 0))

    def full_spec(a):
        s0, s1 = a.shape
        return pl.BlockSpec((s0, s1), lambda i: (0, 0))

    in_specs = [row_spec(FRAME_DIM), row_spec(FRAME_DIM)] + [
        full_spec(w) for w in flat_weights
    ]
    out_specs = [
        row_spec(FRAME_DIM),
        pl.BlockSpec((tile, 1), lambda i: (i, 0)),
        pl.BlockSpec((1, 1), lambda i: (0, 0)),
        pl.BlockSpec((1, 1), lambda i: (0, 0)),
        pl.BlockSpec((1, 1), lambda i: (0, 0)),
        pl.BlockSpec((1, 1), lambda i: (0, 0)),
    ]
    out_shapes = [
        jax.ShapeDtypeStruct((n_rows, FRAME_DIM), jnp.float32),
        jax.ShapeDtypeStruct((n_rows, 1), jnp.int32),
        jax.ShapeDtypeStruct((1, 1), jnp.float32),
        jax.ShapeDtypeStruct((1, 1), jnp.float32),
        jax.ShapeDtypeStruct((1, 1), jnp.float32),
        jax.ShapeDtypeStruct((1, 1), jnp.float32),
    ]
    scratch = [
        pltpu.VMEM((1, NUM_ACTIONS), jnp.float32),
        pltpu.VMEM((1, 1), jnp.float32),
    ]
    fn = functools.partial(_fused_kernel, n_rows, tile)
    return pl.pallas_call(
        fn,
        grid=grid,
        in_specs=in_specs,
        out_specs=out_specs,
        out_shape=out_shapes,
        scratch_shapes=scratch,
    )(ft, ft1, *flat_weights)


def kernel(frame_t, frame_t1, params):
    ei_w, ei_b = params['enc_in']
    eo_w, eo_b = params['enc_out']
    pi_w, pi_b = params['pred_in']
    po_w, po_b = params['pred_out']
    C = params['codebook']

    flat = [ei_w[:FRAME_DIM], ei_w[FRAME_DIM:], ei_b[None, :]]
    for (w, b, g, beta) in params['enc_layers']:
        flat += [w, b[None, :], g[None, :], beta[None, :]]
    flat += [eo_w, eo_b[None, :], C, C.T]
    flat += [pi_w[:FRAME_DIM], pi_w[FRAME_DIM:], pi_b[None, :]]
    for (w, b, g, beta) in params['pred_layers']:
        flat += [w, b[None, :], g[None, :], beta[None, :]]
    flat += [po_w, po_b[None, :]]

    tile = 1024 if frame_t.shape[0] % 1024 == 0 else frame_t.shape[0]
    pred, idx, commit, cb, vq, perp = _run(frame_t, frame_t1, flat, tile)
    return (pred, idx[:, 0],
            commit[0, 0], cb[0, 0], vq[0, 0], perp[0, 0])


# speedup vs baseline: 5.0043x; 1.0479x over previous
import functools

import jax
import jax.numpy as jnp
from jax import lax
from jax.experimental import pallas as pl
from jax.experimental.pallas import tpu as pltpu

FRAME_DIM = 256
HIDDEN = 512
ACTION_DIM = 64
NUM_ACTIONS = 256
COMMIT = 0.25


def _layernorm(x, g, b):
    mu = jnp.mean(x, axis=-1, keepdims=True)
    mu2 = jnp.mean(x * x, axis=-1, keepdims=True)
    r = lax.rsqrt(jnp.maximum(mu2 - mu * mu, 0.0) + 1e-5)
    return (x - mu) * r * g + b


def _gelu(x):
    return x * 0.5 * (1.0 + lax.erf(x * 0.7071067811865476))


def _dot(a, b):
    return jnp.dot(a, b, preferred_element_type=jnp.float32)


def _fused_kernel(n_rows, tile,
                  ft_ref, ft1_ref,
                  ei_wa, ei_wb, ei_b,
                  e0_w, e0_b, e0_g, e0_beta,
                  e1_w, e1_b, e1_g, e1_beta,
                  e2_w, e2_b, e2_g, e2_beta,
                  eo_w, eo_b,
                  c_ref, ct_ref,
                  pi_wa, pi_wb, pi_b,
                  p0_w, p0_b, p0_g, p0_beta,
                  p1_w, p1_b, p1_g, p1_beta,
                  p2_w, p2_b, p2_g, p2_beta,
                  po_w, po_b,
                  pred_ref, idx_ref, commit_ref, cb_ref, vq_ref, perp_ref,
                  hist_acc, sq_acc):
    step = pl.program_id(0)
    nsteps = pl.num_programs(0)

    @pl.when(step == 0)
    def _init():
        hist_acc[...] = jnp.zeros_like(hist_acc)
        sq_acc[...] = jnp.zeros_like(sq_acc)

    ft = ft_ref[...]
    ft1 = ft1_ref[...]

    x = _dot(ft, ei_wa[...]) + _dot(ft1, ei_wb[...]) + ei_b[...]
    for w, b, g, beta in ((e0_w, e0_b, e0_g, e0_beta),
                          (e1_w, e1_b, e1_g, e1_beta),
                          (e2_w, e2_b, e2_g, e2_beta)):
        x = _dot(x, w[...]) + b[...]
        x = _layernorm(x, g[...], beta[...])
        x = _gelu(x)
    z = _dot(x, eo_w[...]) + eo_b[...]

    ct = ct_ref[...]
    c2 = jnp.sum(ct * ct, axis=0, keepdims=True)
    dist = c2 - 2.0 * _dot(z, ct)
    dmin = jnp.min(dist, axis=1, keepdims=True)
    cols = lax.broadcasted_iota(jnp.int32, dist.shape, 1)
    idx = jnp.min(jnp.where(dist == dmin, cols, NUM_ACTIONS), axis=1)
    idx_ref[...] = idx[:, None]
    onehot = (cols == idx[:, None]).astype(jnp.float32)
    quant = _dot(onehot, c_ref[...])

    hist_acc[...] += jnp.sum(onehot, axis=0, keepdims=True)
    diff = z - quant
    sq_acc[...] += jnp.sum(diff * diff)[None, None]

    y = _dot(ft, pi_wa[...]) + _dot(quant, pi_wb[...]) + pi_b[...]
    for w, b, g, beta in ((p0_w, p0_b, p0_g, p0_beta),
                          (p1_w, p1_b, p1_g, p1_beta),
                          (p2_w, p2_b, p2_g, p2_beta)):
        y = _dot(y, w[...]) + b[...]
        y = _layernorm(y, g[...], beta[...])
        y = _gelu(y)
    out = _dot(y, po_w[...]) + po_b[...]
    pred_ref[...] = ft + out

    @pl.when(step == nsteps - 1)
    def _finalize():
        s = sq_acc[...] / (n_rows * ACTION_DIM)
        commit_ref[...] = s
        cb_ref[...] = s
        vq_ref[...] = s + COMMIT * s
        probs = hist_acc[...] / n_rows
        ent = jnp.sum(probs * jnp.log(probs + 1e-10), axis=1, keepdims=True)
        perp_ref[...] = jnp.exp(-ent)


def _run(ft, ft1, flat_weights, tile):
    n_rows = ft.shape[0]
    grid = (n_rows // tile,)

    def row_spec(d):
        return pl.BlockSpec((tile, d), lambda i: (i, 0))

    def full_spec(a):
        s0, s1 = a.shape
        return pl.BlockSpec((s0, s1), lambda i: (0, 0))

    in_specs = [row_spec(FRAME_DIM), row_spec(FRAME_DIM)] + [
        full_spec(w) for w in flat_weights
    ]
    out_specs = [
        row_spec(FRAME_DIM),
        pl.BlockSpec((tile, 1), lambda i: (i, 0)),
        pl.BlockSpec((1, 1), lambda i: (0, 0)),
        pl.BlockSpec((1, 1), lambda i: (0, 0)),
        pl.BlockSpec((1, 1), lambda i: (0, 0)),
        pl.BlockSpec((1, 1), lambda i: (0, 0)),
    ]
    out_shapes = [
        jax.ShapeDtypeStruct((n_rows, FRAME_DIM), jnp.float32),
        jax.ShapeDtypeStruct((n_rows, 1), jnp.int32),
        jax.ShapeDtypeStruct((1, 1), jnp.float32),
        jax.ShapeDtypeStruct((1, 1), jnp.float32),
        jax.ShapeDtypeStruct((1, 1), jnp.float32),
        jax.ShapeDtypeStruct((1, 1), jnp.float32),
    ]
    scratch = [
        pltpu.VMEM((1, NUM_ACTIONS), jnp.float32),
        pltpu.VMEM((1, 1), jnp.float32),
    ]
    fn = functools.partial(_fused_kernel, n_rows, tile)
    return pl.pallas_call(
        fn,
        grid=grid,
        in_specs=in_specs,
        out_specs=out_specs,
        out_shape=out_shapes,
        scratch_shapes=scratch,
    )(ft, ft1, *flat_weights)


def kernel(frame_t, frame_t1, params):
    ei_w, ei_b = params['enc_in']
    eo_w, eo_b = params['enc_out']
    pi_w, pi_b = params['pred_in']
    po_w, po_b = params['pred_out']
    C = params['codebook']

    flat = [ei_w[:FRAME_DIM], ei_w[FRAME_DIM:], ei_b[None, :]]
    for (w, b, g, beta) in params['enc_layers']:
        flat += [w, b[None, :], g[None, :], beta[None, :]]
    flat += [eo_w, eo_b[None, :], C, C.T]
    flat += [pi_w[:FRAME_DIM], pi_w[FRAME_DIM:], pi_b[None, :]]
    for (w, b, g, beta) in params['pred_layers']:
        flat += [w, b[None, :], g[None, :], beta[None, :]]
    flat += [po_w, po_b[None, :]]

    tile = 1024 if frame_t.shape[0] % 1024 == 0 else frame_t.shape[0]
    pred, idx, commit, cb, vq, perp = _run(frame_t, frame_t1, flat, tile)
    return (pred, idx[:, 0],
            commit[0, 0], cb[0, 0], vq[0, 0], perp[0, 0])
